# baseline (speedup 1.0000x reference)
import functools

import jax
import jax.numpy as jnp
from jax import lax
from jax.experimental import pallas as pl
from jax.experimental.pallas import tpu as pltpu

_B = 16
_T = 400
_H = 512
_EPS = 1e-5
_ROWS = _B * _T
_PAD_ROWS = _ROWS + 16


_CH = 16
_R = _ROWS // _CH


def _chunk_stats(y, t_valid):
    s1 = None
    s2 = None
    for bl in range(_R // _T):
        v = y[bl * _T:bl * _T + t_valid]
        a = jnp.sum(v, axis=0, keepdims=True)
        q = jnp.sum(v * v, axis=0, keepdims=True)
        s1 = a if s1 is None else s1 + a
        s2 = q if s2 is None else s2 + q
    return s1, s2


def _affine(s1, s2, g, beta, count):
    mean = s1 * (1.0 / count)
    ex2 = s2 * (1.0 / count)
    var = ex2 - mean * mean
    scale = g * lax.rsqrt(var + _EPS)
    shift = beta - mean * scale
    return scale, shift


def _norm_chunk(ref, scale, shift, c):
    lo = c * _R
    ref[lo:lo + _R] = jnp.clip(ref[lo:lo + _R] * scale + shift, 0.0, 20.0)


def _stage(in_ref, out_ref, t_valid, dot_chunk, in_affine):
    if in_affine is not None:
        _norm_chunk(in_ref, *in_affine, 0)
        _norm_chunk(in_ref, *in_affine, 1)
    s1 = None
    s2 = None
    for c in range(_CH):
        if in_affine is not None and c + 2 < _CH:
            _norm_chunk(in_ref, *in_affine, c + 2)
        y = dot_chunk(c)
        a, q = _chunk_stats(y, t_valid)
        s1 = a if s1 is None else s1 + a
        s2 = q if s2 is None else s2 + q
        m_c = y.shape[0]
        out_ref[c * _R:c * _R + m_c] = y
    return s1, s2


def _mega_kernel(x_ref, w1_ref, b1_ref, g1_ref, t1_ref,
                 w2_ref, b2_ref, g2_ref, t2_ref,
                 w3_ref, b3_ref, g3_ref, t3_ref,
                 w4_ref, b4_ref, g4_ref, t4_ref,
                 w5_ref, b5_ref, g5_ref, t5_ref,
                 m_ref, a_ref, b_ref, stg_ref, sem_ref):
    f32 = jnp.float32

    def conv_chunk(in_ref, w_ref, bias_ref, taps, dil, c, last_m):
        lo = c * _R
        m_c = _R if c < _CH - 1 else last_m
        acc = jnp.dot(in_ref[lo:lo + m_c], w_ref[0],
                      preferred_element_type=f32)
        for k in range(1, taps):
            acc = acc + jnp.dot(in_ref[lo + k * dil:lo + k * dil + m_c],
                                w_ref[k], preferred_element_type=f32)
        return acc + bias_ref[...]

    def lin_chunk(in_ref, w_ref, bias_ref, c):
        lo = c * _R
        return lax.dot_general(
            in_ref[lo:lo + _R], w_ref[...], (((1,), (1,)), ((), ())),
            preferred_element_type=f32) + bias_ref[...]

    s1, s2 = _stage(x_ref, a_ref, 396,
                    lambda c: conv_chunk(x_ref, w1_ref, b1_ref, 5, 1, c,
                                         _R - 4),
                    None)
    af1 = _affine(s1, s2, g1_ref[...], t1_ref[...], _B * 396.0)

    s1, s2 = _stage(a_ref, b_ref, 392,
                    lambda c: conv_chunk(a_ref, w2_ref, b2_ref, 3, 2, c, _R),
                    af1)
    af2 = _affine(s1, s2, g2_ref[...], t2_ref[...], _B * 392.0)

    s1, s2 = _stage(b_ref, a_ref, 384,
                    lambda c: conv_chunk(b_ref, w3_ref, b3_ref, 3, 4, c, _R),
                    af2)
    af3 = _affine(s1, s2, g3_ref[...], t3_ref[...], _B * 384.0)

    s1, s2 = _stage(a_ref, b_ref, 384,
                    lambda c: lin_chunk(a_ref, w4_ref, b4_ref, c),
                    af3)
    af4 = _affine(s1, s2, g4_ref[...], t4_ref[...], _B * 384.0)

    s1, s2 = _stage(b_ref, a_ref, 384,
                    lambda c: lin_chunk(b_ref, w5_ref, b5_ref, c),
                    af4)
    af5 = _affine(s1, s2, g5_ref[...], t5_ref[...], _B * 384.0)

    bpc = _R // _T
    _norm_chunk(a_ref, *af5, 0)
    for bb in range(_B):
        if bb % bpc == 0 and bb // bpc + 1 < _CH:
            _norm_chunk(a_ref, *af5, bb // bpc + 1)
        s = bb % 2
        if bb >= 2:
            pltpu.make_async_copy(stg_ref.at[s], m_ref.at[bb - 2],
                                  sem_ref.at[s]).wait()
        lo = bb * _T
        stg_ref[s] = lax.dot_general(
            b_ref[lo:lo + 384], a_ref[lo:lo + 384], (((0,), (0,)), ((), ())),
            preferred_element_type=f32)
        pltpu.make_async_copy(stg_ref.at[s], m_ref.at[bb],
                              sem_ref.at[s]).start()
    pltpu.make_async_copy(stg_ref.at[0], m_ref.at[_B - 2],
                          sem_ref.at[0]).wait()
    pltpu.make_async_copy(stg_ref.at[1], m_ref.at[_B - 1],
                          sem_ref.at[1]).wait()


def _bot_kernel(m_ref, w_ref, bb_ref, g_ref, bta_ref, ew_ref, eb_ref,
                out_ref, acc_ref, *, n_steps):
    k = pl.program_id(0)

    @pl.when(k == 0)
    def _():
        acc_ref[...] = jnp.zeros_like(acc_ref)

    acc_ref[...] += lax.dot_general(
        m_ref[...], w_ref[...], (((1,), (1,)), ((), ())),
        preferred_element_type=jnp.float32)

    @pl.when(k == n_steps - 1)
    def _():
        h = acc_ref[...] + bb_ref[...]
        mean = jnp.mean(h, axis=0, keepdims=True)
        var = jnp.mean(h * h, axis=0, keepdims=True) - mean * mean
        hn = jnp.clip(g_ref[...] * (h - mean) * lax.rsqrt(var + _EPS)
                      + bta_ref[...], 0.0, 20.0)
        emb = lax.dot_general(hn, ew_ref[...], (((1,), (1,)), ((), ())),
                              preferred_element_type=jnp.float32)
        emb = emb + eb_ref[...]
        inv = lax.rsqrt(jnp.sum(emb * emb, axis=1, keepdims=True) + 1e-10)
        out_ref[...] = emb * inv * 10.0


def _row(v):
    return v.reshape(1, -1)


def _vm():
    return pl.BlockSpec(memory_space=pltpu.VMEM)


def _full(shape):
    return pl.BlockSpec(shape, lambda k: tuple(0 for _ in shape))


def kernel(input_x, conv1_w, conv1_b, bn1_g, bn1_b, conv2_w, conv2_b, bn2_g,
           bn2_b, conv3_w, conv3_b, bn3_g, bn3_b, lin4_w, lin4_b, bn4_g,
           bn4_b, lin5_w, lin5_b, bn5_g, bn5_b, bot_w, bot_b, bnb_g, bnb_b,
           emb_w, emb_b):
    f32 = jnp.float32
    b, h = _B, _H
    xs = input_x[:, 0].reshape(_ROWS, input_x.shape[3])

    w1 = conv1_w.transpose(2, 1, 0)
    w2 = conv2_w.transpose(2, 1, 0)
    w3 = conv3_w.transpose(2, 1, 0)
    w4 = lin4_w
    w5 = lin5_w

    args = [xs,
            w1, _row(conv1_b), _row(bn1_g), _row(bn1_b),
            w2, _row(conv2_b), _row(bn2_g), _row(bn2_b),
            w3, _row(conv3_b), _row(bn3_g), _row(bn3_b),
            w4, _row(lin4_b), _row(bn4_g), _row(bn4_b),
            w5, _row(lin5_b), _row(bn5_g), _row(bn5_b)]

    m = pl.pallas_call(
        _mega_kernel,
        in_specs=[_vm() for _ in args],
        out_specs=pl.BlockSpec(memory_space=pl.ANY),
        out_shape=jax.ShapeDtypeStruct((b, h, h), f32),
        scratch_shapes=[
            pltpu.VMEM((_PAD_ROWS, h), f32),
            pltpu.VMEM((_PAD_ROWS, h), f32),
            pltpu.VMEM((2, h, h), f32),
            pltpu.SemaphoreType.DMA((2,)),
        ],
        compiler_params=pltpu.CompilerParams(
            vmem_limit_bytes=58 * 1024 * 1024),
        name="mega",
    )(*args)

    msq = bot_w.shape[1]
    m2 = m.reshape(b, msq)
    ck = 8192
    n_steps = msq // ck
    out = pl.pallas_call(
        functools.partial(_bot_kernel, n_steps=n_steps),
        grid=(n_steps,),
        in_specs=[
            pl.BlockSpec((b, ck), lambda k: (0, k)),
            pl.BlockSpec((h, ck), lambda k: (0, k)),
            _full((1, h)), _full((1, h)), _full((1, h)), _full((h, h)),
            _full((1, h)),
        ],
        out_specs=pl.BlockSpec((b, h), lambda k: (0, 0)),
        out_shape=jax.ShapeDtypeStruct((b, h), f32),
        scratch_shapes=[pltpu.VMEM((b, h), f32)],
        compiler_params=pltpu.CompilerParams(
            dimension_semantics=("arbitrary",),
            vmem_limit_bytes=52 * 1024 * 1024),
        name="bot",
    )(m2, bot_w, _row(bot_b), _row(bnb_g), _row(bnb_b), emb_w, _row(emb_b))
    return out


# speedup vs baseline: 1.0007x; 1.0007x over previous
import functools

import jax
import jax.numpy as jnp
from jax import lax
from jax.experimental import pallas as pl
from jax.experimental.pallas import tpu as pltpu

_B = 16
_T = 400
_H = 512
_EPS = 1e-5
_ROWS = _B * _T
_PAD_ROWS = _ROWS + 16


_CH = 8
_R = _ROWS // _CH


def _chunk_stats(y, t_valid):
    s1 = None
    s2 = None
    for bl in range(_R // _T):
        v = y[bl * _T:bl * _T + t_valid]
        a = jnp.sum(v, axis=0, keepdims=True)
        q = jnp.sum(v * v, axis=0, keepdims=True)
        s1 = a if s1 is None else s1 + a
        s2 = q if s2 is None else s2 + q
    return s1, s2


def _affine(s1, s2, g, beta, count):
    mean = s1 * (1.0 / count)
    ex2 = s2 * (1.0 / count)
    var = ex2 - mean * mean
    scale = g * lax.rsqrt(var + _EPS)
    shift = beta - mean * scale
    return scale, shift


def _norm_chunk(ref, scale, shift, c):
    lo = c * _R
    ref[lo:lo + _R] = jnp.clip(ref[lo:lo + _R] * scale + shift, 0.0, 20.0)


def _stage(in_ref, out_ref, t_valid, dot_chunk, in_affine):
    if in_affine is not None:
        _norm_chunk(in_ref, *in_affine, 0)
        _norm_chunk(in_ref, *in_affine, 1)
    s1 = None
    s2 = None
    for c in range(_CH):
        if in_affine is not None and c + 2 < _CH:
            _norm_chunk(in_ref, *in_affine, c + 2)
        y = dot_chunk(c)
        a, q = _chunk_stats(y, t_valid)
        s1 = a if s1 is None else s1 + a
        s2 = q if s2 is None else s2 + q
        m_c = y.shape[0]
        out_ref[c * _R:c * _R + m_c] = y
    return s1, s2


def _mega_kernel(x_ref, w1_ref, b1_ref, g1_ref, t1_ref,
                 w2_ref, b2_ref, g2_ref, t2_ref,
                 w3_ref, b3_ref, g3_ref, t3_ref,
                 w4_ref, b4_ref, g4_ref, t4_ref,
                 w5_ref, b5_ref, g5_ref, t5_ref,
                 m_ref, a_ref, b_ref, stg_ref, sem_ref):
    f32 = jnp.float32

    def conv_chunk(in_ref, w_ref, bias_ref, taps, dil, c, last_m):
        lo = c * _R
        m_c = _R if c < _CH - 1 else last_m
        acc = jnp.dot(in_ref[lo:lo + m_c], w_ref[0],
                      preferred_element_type=f32)
        for k in range(1, taps):
            acc = acc + jnp.dot(in_ref[lo + k * dil:lo + k * dil + m_c],
                                w_ref[k], preferred_element_type=f32)
        return acc + bias_ref[...]

    def lin_chunk(in_ref, w_ref, bias_ref, c):
        lo = c * _R
        return lax.dot_general(
            in_ref[lo:lo + _R], w_ref[...], (((1,), (1,)), ((), ())),
            preferred_element_type=f32) + bias_ref[...]

    s1, s2 = _stage(x_ref, a_ref, 396,
                    lambda c: conv_chunk(x_ref, w1_ref, b1_ref, 5, 1, c,
                                         _R - 4),
                    None)
    af1 = _affine(s1, s2, g1_ref[...], t1_ref[...], _B * 396.0)

    s1, s2 = _stage(a_ref, b_ref, 392,
                    lambda c: conv_chunk(a_ref, w2_ref, b2_ref, 3, 2, c, _R),
                    af1)
    af2 = _affine(s1, s2, g2_ref[...], t2_ref[...], _B * 392.0)

    s1, s2 = _stage(b_ref, a_ref, 384,
                    lambda c: conv_chunk(b_ref, w3_ref, b3_ref, 3, 4, c, _R),
                    af2)
    af3 = _affine(s1, s2, g3_ref[...], t3_ref[...], _B * 384.0)

    s1, s2 = _stage(a_ref, b_ref, 384,
                    lambda c: lin_chunk(a_ref, w4_ref, b4_ref, c),
                    af3)
    af4 = _affine(s1, s2, g4_ref[...], t4_ref[...], _B * 384.0)

    s1, s2 = _stage(b_ref, a_ref, 384,
                    lambda c: lin_chunk(b_ref, w5_ref, b5_ref, c),
                    af4)
    af5 = _affine(s1, s2, g5_ref[...], t5_ref[...], _B * 384.0)

    bpc = _R // _T
    _norm_chunk(a_ref, *af5, 0)
    for bb in range(_B):
        if bb % bpc == 0 and bb // bpc + 1 < _CH:
            _norm_chunk(a_ref, *af5, bb // bpc + 1)
        s = bb % 2
        if bb >= 2:
            pltpu.make_async_copy(stg_ref.at[s], m_ref.at[bb - 2],
                                  sem_ref.at[s]).wait()
        lo = bb * _T
        stg_ref[s] = lax.dot_general(
            b_ref[lo:lo + 384], a_ref[lo:lo + 384], (((0,), (0,)), ((), ())),
            preferred_element_type=f32)
        pltpu.make_async_copy(stg_ref.at[s], m_ref.at[bb],
                              sem_ref.at[s]).start()
    pltpu.make_async_copy(stg_ref.at[0], m_ref.at[_B - 2],
                          sem_ref.at[0]).wait()
    pltpu.make_async_copy(stg_ref.at[1], m_ref.at[_B - 1],
                          sem_ref.at[1]).wait()


def _bot_kernel(m_ref, w_ref, bb_ref, g_ref, bta_ref, ew_ref, eb_ref,
                out_ref, acc_ref, *, n_steps):
    k = pl.program_id(0)

    @pl.when(k == 0)
    def _():
        acc_ref[...] = jnp.zeros_like(acc_ref)

    acc_ref[...] += lax.dot_general(
        m_ref[...], w_ref[...], (((1,), (1,)), ((), ())),
        preferred_element_type=jnp.float32)

    @pl.when(k == n_steps - 1)
    def _():
        h = acc_ref[...] + bb_ref[...]
        mean = jnp.mean(h, axis=0, keepdims=True)
        var = jnp.mean(h * h, axis=0, keepdims=True) - mean * mean
        hn = jnp.clip(g_ref[...] * (h - mean) * lax.rsqrt(var + _EPS)
                      + bta_ref[...], 0.0, 20.0)
        emb = lax.dot_general(hn, ew_ref[...], (((1,), (1,)), ((), ())),
                              preferred_element_type=jnp.float32)
        emb = emb + eb_ref[...]
        inv = lax.rsqrt(jnp.sum(emb * emb, axis=1, keepdims=True) + 1e-10)
        out_ref[...] = emb * inv * 10.0


def _row(v):
    return v.reshape(1, -1)


def _vm():
    return pl.BlockSpec(memory_space=pltpu.VMEM)


def _full(shape):
    return pl.BlockSpec(shape, lambda k: tuple(0 for _ in shape))


def kernel(input_x, conv1_w, conv1_b, bn1_g, bn1_b, conv2_w, conv2_b, bn2_g,
           bn2_b, conv3_w, conv3_b, bn3_g, bn3_b, lin4_w, lin4_b, bn4_g,
           bn4_b, lin5_w, lin5_b, bn5_g, bn5_b, bot_w, bot_b, bnb_g, bnb_b,
           emb_w, emb_b):
    f32 = jnp.float32
    b, h = _B, _H
    xs = input_x[:, 0].reshape(_ROWS, input_x.shape[3])

    w1 = conv1_w.transpose(2, 1, 0)
    w2 = conv2_w.transpose(2, 1, 0)
    w3 = conv3_w.transpose(2, 1, 0)
    w4 = lin4_w
    w5 = lin5_w

    args = [xs,
            w1, _row(conv1_b), _row(bn1_g), _row(bn1_b),
            w2, _row(conv2_b), _row(bn2_g), _row(bn2_b),
            w3, _row(conv3_b), _row(bn3_g), _row(bn3_b),
            w4, _row(lin4_b), _row(bn4_g), _row(bn4_b),
            w5, _row(lin5_b), _row(bn5_g), _row(bn5_b)]

    m = pl.pallas_call(
        _mega_kernel,
        in_specs=[_vm() for _ in args],
        out_specs=pl.BlockSpec(memory_space=pl.ANY),
        out_shape=jax.ShapeDtypeStruct((b, h, h), f32),
        scratch_shapes=[
            pltpu.VMEM((_PAD_ROWS, h), f32),
            pltpu.VMEM((_PAD_ROWS, h), f32),
            pltpu.VMEM((2, h, h), f32),
            pltpu.SemaphoreType.DMA((2,)),
        ],
        compiler_params=pltpu.CompilerParams(
            vmem_limit_bytes=58 * 1024 * 1024),
        name="mega",
    )(*args)

    msq = bot_w.shape[1]
    m2 = m.reshape(b, msq)
    ck = 8192
    n_steps = msq // ck
    out = pl.pallas_call(
        functools.partial(_bot_kernel, n_steps=n_steps),
        grid=(n_steps,),
        in_specs=[
            pl.BlockSpec((b, ck), lambda k: (0, k)),
            pl.BlockSpec((h, ck), lambda k: (0, k)),
            _full((1, h)), _full((1, h)), _full((1, h)), _full((h, h)),
            _full((1, h)),
        ],
        out_specs=pl.BlockSpec((b, h), lambda k: (0, 0)),
        out_shape=jax.ShapeDtypeStruct((b, h), f32),
        scratch_shapes=[pltpu.VMEM((b, h), f32)],
        compiler_params=pltpu.CompilerParams(
            dimension_semantics=("arbitrary",),
            vmem_limit_bytes=52 * 1024 * 1024),
        name="bot",
    )(m2, bot_w, _row(bot_b), _row(bnb_g), _row(bnb_b), emb_w, _row(emb_b))
    return out
